# Initial kernel scaffold; baseline (speedup 1.0000x reference)
import jax
import jax.numpy as jnp
from jax.experimental import pallas as pl


def kernel(x, pre_encode_b, W, WT, b1, b2):
    raise NotImplementedError("write your pallas kernel here")


# speedup vs baseline: 6.8700x; 6.8700x over previous
import functools

import jax
import jax.numpy as jnp
from jax import lax
from jax.experimental import pallas as pl
from jax.experimental.pallas import tpu as pltpu

TOKENS = 4096
INPUT_SIZE = 2048
HIDDEN_SIZE = 2048
K = 32
BT = 256


def _sae_block(x_ref, peb_ref, wt_ref, w_ref, b1_ref, b2_ref, o_ref, work_ref):
    xc = x_ref[...] - peb_ref[...]
    pre = (
        jnp.dot(xc, wt_ref[...], preferred_element_type=jnp.float32)
        + b1_ref[...]
    )
    work_ref[...] = pre

    col = lax.broadcasted_iota(jnp.int32, pre.shape, 1)
    neg_inf = jnp.float32(float("-inf"))

    def body(_, carry):
        work = work_ref[...]
        m = jnp.max(work, axis=1, keepdims=True)
        c = jnp.min(jnp.where(work == m, col, HIDDEN_SIZE), axis=1, keepdims=True)
        work_ref[...] = jnp.where(col == c, neg_inf, work)
        return carry

    lax.fori_loop(0, K, body, 0)

    a = jnp.where(work_ref[...] == neg_inf, pre, jnp.float32(0.0))
    o_ref[...] = (
        jnp.dot(a, w_ref[...], preferred_element_type=jnp.float32) + b2_ref[...]
    )


@jax.jit
def _sae_fused(x, peb2, WT, W, b12, b22):
    grid = (TOKENS // BT,)
    return pl.pallas_call(
        _sae_block,
        grid=grid,
        in_specs=[
            pl.BlockSpec((BT, INPUT_SIZE), lambda i: (i, 0)),
            pl.BlockSpec((1, HIDDEN_SIZE), lambda i: (0, 0)),
            pl.BlockSpec((INPUT_SIZE, HIDDEN_SIZE), lambda i: (0, 0)),
            pl.BlockSpec((HIDDEN_SIZE, INPUT_SIZE), lambda i: (0, 0)),
            pl.BlockSpec((1, HIDDEN_SIZE), lambda i: (0, 0)),
            pl.BlockSpec((1, INPUT_SIZE), lambda i: (0, 0)),
        ],
        out_specs=pl.BlockSpec((BT, INPUT_SIZE), lambda i: (i, 0)),
        out_shape=jax.ShapeDtypeStruct((TOKENS, INPUT_SIZE), jnp.float32),
        scratch_shapes=[pltpu.VMEM((BT, HIDDEN_SIZE), jnp.float32)],
        compiler_params=pltpu.CompilerParams(
            dimension_semantics=("arbitrary",),
        ),
    )(x, peb2, WT, W, b12, b22)


def kernel(x, pre_encode_b, W, WT, b1, b2):
    peb2 = pre_encode_b.reshape(1, HIDDEN_SIZE)
    b12 = b1.reshape(1, HIDDEN_SIZE)
    b22 = b2.reshape(1, INPUT_SIZE)
    return _sae_fused(x, peb2, WT, W, b12, b22)


# speedup vs baseline: 10.4689x; 1.5239x over previous
import functools

import jax
import jax.numpy as jnp
from jax import lax
from jax.experimental import pallas as pl
from jax.experimental.pallas import tpu as pltpu

TOKENS = 4096
INPUT_SIZE = 2048
HIDDEN_SIZE = 2048
K = 32
BT = 256


def _sae_block(x_ref, peb_ref, wt_ref, w_ref, b1_ref, b2_ref, o_ref):
    xc = x_ref[...] - peb_ref[...]
    pre = (
        jnp.dot(xc, wt_ref[...], preferred_element_type=jnp.float32)
        + b1_ref[...]
    )

    bits = lax.bitcast_convert_type(pre, jnp.int32)
    s = bits ^ (jnp.int32(0x7FFFFFFF) & (bits >> 31))
    min_int = jnp.int32(-(2**31))

    def body(i, o):
        bit = lax.shift_left(jnp.int32(1), 31 - i)
        cand_o = o | bit
        cand_s = cand_o ^ min_int
        cnt = jnp.sum(
            jnp.where(s >= cand_s, jnp.int32(1), jnp.int32(0)),
            axis=1,
            keepdims=True,
        )
        return jnp.where(cnt >= K, cand_o, o)

    o = lax.fori_loop(0, 32, body, jnp.zeros((pre.shape[0], 1), jnp.int32))
    thr = o ^ min_int

    a = jnp.where(s >= thr, pre, jnp.float32(0.0))
    o_ref[...] = (
        jnp.dot(a, w_ref[...], preferred_element_type=jnp.float32) + b2_ref[...]
    )


@jax.jit
def _sae_fused(x, peb2, WT, W, b12, b22):
    grid = (TOKENS // BT,)
    return pl.pallas_call(
        _sae_block,
        grid=grid,
        in_specs=[
            pl.BlockSpec((BT, INPUT_SIZE), lambda i: (i, 0)),
            pl.BlockSpec((1, HIDDEN_SIZE), lambda i: (0, 0)),
            pl.BlockSpec((INPUT_SIZE, HIDDEN_SIZE), lambda i: (0, 0)),
            pl.BlockSpec((HIDDEN_SIZE, INPUT_SIZE), lambda i: (0, 0)),
            pl.BlockSpec((1, HIDDEN_SIZE), lambda i: (0, 0)),
            pl.BlockSpec((1, INPUT_SIZE), lambda i: (0, 0)),
        ],
        out_specs=pl.BlockSpec((BT, INPUT_SIZE), lambda i: (i, 0)),
        out_shape=jax.ShapeDtypeStruct((TOKENS, INPUT_SIZE), jnp.float32),
        compiler_params=pltpu.CompilerParams(
            dimension_semantics=("arbitrary",),
        ),
    )(x, peb2, WT, W, b12, b22)


def kernel(x, pre_encode_b, W, WT, b1, b2):
    peb2 = pre_encode_b.reshape(1, HIDDEN_SIZE)
    b12 = b1.reshape(1, HIDDEN_SIZE)
    b22 = b2.reshape(1, INPUT_SIZE)
    return _sae_fused(x, peb2, WT, W, b12, b22)
